# Initial kernel scaffold; baseline (speedup 1.0000x reference)
import jax
import jax.numpy as jnp
from jax.experimental import pallas as pl


def kernel(x, edge_index, W0, b0, W1, b1, W2, b2, W3, b3, W4, b4, Wm, bm, p):
    raise NotImplementedError("write your pallas kernel here")


# speedup vs baseline: 1.0001x; 1.0001x over previous
import jax
import jax.numpy as jnp
from jax.experimental import pallas as pl

N = 10000
K = 20
ALPHA = [0.7, 0.7, 0.7 / 2.0, 0.7 / 3.0, 0.7 / 4.0]


def _identity_kernel(x_ref, o_ref):
    o_ref[...] = x_ref[...]


def _ssg(x, row, col, norm, W, b, alpha):
    h = x * alpha
    for _ in range(K):
        x = jnp.zeros_like(x).at[col].add(norm[:, None] * x[row])
        h = h + (1.0 - alpha) / K * x
    return h @ W + b


def kernel(x, edge_index, W0, b0, W1, b1, W2, b2, W3, b3, W4, b4, Wm, bm, p):
    loop = jnp.arange(N, dtype=edge_index.dtype)
    row = jnp.concatenate([edge_index[0], loop])
    col = jnp.concatenate([edge_index[1], loop])
    deg = jnp.zeros((N,), jnp.float32).at[col].add(1.0)
    dinv = jax.lax.rsqrt(jnp.maximum(deg, 1e-12))
    norm = dinv[row] * dinv[col]
    h = x[:, :4]
    h = jax.nn.relu(_ssg(h, row, col, norm, W0, b0, ALPHA[0]))
    h = jax.nn.relu(_ssg(h, row, col, norm, W1, b1, ALPHA[1]))
    h = jax.nn.relu(_ssg(h, row, col, norm, W2, b2, ALPHA[2])) + h
    h = jax.nn.relu(_ssg(h, row, col, norm, W3, b3, ALPHA[3])) + h
    h = jax.nn.relu(_ssg(h, row, col, norm, W4, b4, ALPHA[4])) + h
    h = h @ Wm + bm
    score = jnp.tanh(h @ p / jnp.linalg.norm(p))
    vals, perm = jax.lax.top_k(score, 30)
    hk = h[perm] * vals[:, None]
    out = jnp.max(hk, axis=0, keepdims=True)
    out = pl.pallas_call(
        _identity_kernel,
        out_shape=jax.ShapeDtypeStruct(out.shape, out.dtype),
    )(out)
    return out


# speedup vs baseline: 5.5429x; 5.5421x over previous
import jax
import jax.numpy as jnp
from jax import lax
from jax.experimental import pallas as pl
from jax.experimental.pallas import tpu as pltpu
from jax.experimental.pallas import tpu_sc as plsc

N = 10000
E = 320000
K = 20
ALPHA = [0.7, 0.7, 0.7 / 2.0, 0.7 / 3.0, 0.7 / 4.0]

NPAD = 10240
NC, NS = 2, 16
NW = NC * NS
CH = 128
NCHUNK = 79
EPT = NCHUNK * CH
EPAD = NW * EPT
RPS = NPAD // NS
DUMMY = NPAD - 1
WF = 128

_mesh = plsc.VectorSubcoreMesh(core_axis_name="c", subcore_axis_name="s")


def _sc_hop_body(z_hbm, rowt, colt, ypart, row_v, col_v, bufa, y_sh):
    c = lax.axis_index("c")
    s = lax.axis_index("s")
    wid = c * NS + s
    pltpu.sync_copy(rowt.at[wid], row_v)
    pltpu.sync_copy(colt.at[wid], col_v)

    zero16 = jnp.zeros((16,), jnp.float32)

    @pl.loop(0, CH)
    def _zero(i):
        for g in range(WF // 16):
            bufa[i, pl.ds(g * 16, 16)] = zero16

    for q in range(RPS // CH):
        pltpu.sync_copy(bufa, y_sh.at[pl.ds(s * RPS + q * CH, CH)])
    plsc.subcore_barrier()

    @pl.loop(0, NCHUNK)
    def _edges(j):
        pltpu.sync_copy(z_hbm.at[row_v.at[j]], bufa)
        pltpu.sync_copy(bufa, y_sh.at[col_v.at[j]], add=True)

    plsc.subcore_barrier()

    for q in range(RPS // CH):
        off = s * RPS + q * CH
        pltpu.sync_copy(y_sh.at[pl.ds(off, CH)], bufa)
        pltpu.sync_copy(bufa, ypart.at[c].at[pl.ds(off, CH)])


_sc_hop = pl.kernel(
    _sc_hop_body,
    out_type=jax.ShapeDtypeStruct((NC, NPAD, WF), jnp.float32),
    mesh=_mesh,
    scratch_types=[
        pltpu.VMEM((NCHUNK, CH), jnp.int32),
        pltpu.VMEM((NCHUNK, CH), jnp.int32),
        pltpu.VMEM((CH, WF), jnp.float32),
        pltpu.VMEM_SHARED((NPAD, WF), jnp.float32),
    ],
)


_RB = 1280
_GRID = NPAD // _RB


def _prep_body(dp0, dp1, x4, dvb, dv2b, h0, z0):
    b = pl.program_id(0)
    deg = dp0[...][:, 0:1] + dp1[...][:, 0:1] + 1.0
    dv = lax.rsqrt(deg)
    ridx = lax.broadcasted_iota(jnp.int32, (_RB, 1), 0) + b * _RB
    dv = jnp.where(ridx < N, dv, 0.0)
    dvb[...] = jnp.broadcast_to(dv, (_RB, WF))
    dv2b[...] = jnp.broadcast_to(dv * dv, (_RB, WF))
    h0[...] = ALPHA[0] * x4[...]
    z0[...] = dv * x4[...]


_prep = pl.pallas_call(
    _prep_body,
    grid=(_GRID,),
    in_specs=[pl.BlockSpec((_RB, WF), lambda b: (b, 0))] * 3,
    out_specs=[pl.BlockSpec((_RB, WF), lambda b: (b, 0))] * 4,
    out_shape=[jax.ShapeDtypeStruct((NPAD, WF), jnp.float32)] * 4,
)


def _make_combine(cst):
    def body(y0, y1, z, h, dvb, dv2b, ho, zo):
        yt = y0[...] + y1[...] + z[...]
        ho[...] = h[...] + cst * (dvb[...] * yt)
        zo[...] = dv2b[...] * yt

    return pl.pallas_call(
        body,
        grid=(_GRID,),
        in_specs=[pl.BlockSpec((_RB, WF), lambda b: (b, 0))] * 6,
        out_specs=[pl.BlockSpec((_RB, WF), lambda b: (b, 0))] * 2,
        out_shape=[jax.ShapeDtypeStruct((NPAD, WF), jnp.float32)] * 2,
    )


def _make_layer_end(alpha_next, has_res):
    def body(*refs):
        if has_res:
            h, W, bvec, res, dvb, xo, ho, zo = refs
        else:
            h, W, bvec, dvb, xo, ho, zo = refs
        v = jnp.dot(h[...], W[...], preferred_element_type=jnp.float32)
        v = jnp.maximum(v + bvec[...], 0.0)
        if has_res:
            v = v + res[...]
        xo[...] = v
        ho[...] = alpha_next * v
        zo[...] = dvb[...] * v

    in_specs = [
        pl.BlockSpec((_RB, WF), lambda b: (b, 0)),
        pl.BlockSpec((WF, 128), lambda b: (0, 0)),
        pl.BlockSpec((1, 128), lambda b: (0, 0)),
    ]
    if has_res:
        in_specs.append(pl.BlockSpec((_RB, 128), lambda b: (b, 0)))
    in_specs.append(pl.BlockSpec((_RB, 128), lambda b: (b, 0)))
    return pl.pallas_call(
        body,
        grid=(_GRID,),
        in_specs=in_specs,
        out_specs=[pl.BlockSpec((_RB, 128), lambda b: (b, 0))] * 3,
        out_shape=[jax.ShapeDtypeStruct((NPAD, 128), jnp.float32)] * 3,
    )


def _head_body(h, Wm, bm, pvec, o):
    hm = jnp.dot(h[...], Wm[...], preferred_element_type=jnp.float32) + bm[...]
    pv = pvec[...]
    pn = pv * lax.rsqrt(jnp.sum(pv * pv))
    s = jnp.tanh(jnp.sum(hm * pn, axis=1, keepdims=True))
    ridx = lax.broadcasted_iota(jnp.int32, (NPAD, 1), 0)
    s = jnp.where(ridx < N, s, -jnp.inf)
    acc0 = jnp.full((1, 128), -jnp.inf, jnp.float32)

    def it(_, carry):
        sc, acc = carry
        m = jnp.max(sc)
        mask = sc == m
        contrib = jnp.max(jnp.where(mask, m * hm, -jnp.inf), axis=0,
                          keepdims=True)
        acc = jnp.maximum(acc, contrib)
        sc = jnp.where(mask, -jnp.inf, sc)
        return sc, acc

    _, acc = lax.fori_loop(0, 30, it, (s, acc0))
    o[...] = acc


_head = pl.pallas_call(
    _head_body,
    out_shape=jax.ShapeDtypeStruct((1, 128), jnp.float32),
)


def kernel(x, edge_index, W0, b0, W1, b1, W2, b2, W3, b3, W4, b4, Wm, bm, p):
    row = edge_index[0].astype(jnp.int32)
    col = edge_index[1].astype(jnp.int32)
    fill = jnp.full((EPAD - E,), DUMMY, jnp.int32)
    rowt = jnp.concatenate([row, fill]).reshape(NW, NCHUNK, CH)
    colt = jnp.concatenate([col, fill]).reshape(NW, NCHUNK, CH)

    ones_tab = jnp.ones((NPAD, WF), jnp.float32)
    degp = _sc_hop(ones_tab, rowt, colt)
    x4 = jnp.pad(x[:, :4], ((0, NPAD - N), (0, WF - 4)))
    dvb, dv2b, h, z = _prep(degp[0], degp[1], x4)

    W0p = jnp.pad(W0, ((0, WF - 4), (0, 0)))
    Ws = [W0p, W1, W2, W3, W4]
    bs = [b0.reshape(1, 128), b1.reshape(1, 128), b2.reshape(1, 128),
          b3.reshape(1, 128), b4.reshape(1, 128)]

    def run_layer(h, z, comb):
        def step(_, hz):
            h, z = hz
            yp = _sc_hop(z, rowt, colt)
            h, z = comb(yp[0], yp[1], z, h, dvb, dv2b)
            return h, z

        return lax.fori_loop(0, K, step, (h, z))

    xs = [None]
    for li in range(5):
        h, z = run_layer(h, z, _make_combine((1 - ALPHA[li]) / K))
        alpha_next = ALPHA[li + 1] if li < 4 else 0.0
        has_res = li >= 2
        if has_res:
            xn, h, z = _make_layer_end(alpha_next, True)(
                h, Ws[li], bs[li], xs[li], dvb)
        else:
            xn, h, z = _make_layer_end(alpha_next, False)(
                h, Ws[li], bs[li], dvb)
        xs.append(xn)

    return _head(xs[5], Wm, bm.reshape(1, 128), p.reshape(1, 128))


# speedup vs baseline: 6.4122x; 1.1568x over previous
import jax
import jax.numpy as jnp
from jax import lax
from jax.experimental import pallas as pl
from jax.experimental.pallas import tpu as pltpu
from jax.experimental.pallas import tpu_sc as plsc

N = 10000
E = 320000
K = 20
ALPHA = [0.7, 0.7, 0.7 / 2.0, 0.7 / 3.0, 0.7 / 4.0]

NPAD = 10240
NC, NS = 2, 16
CH = 128
HF = 64
NPH = 2
CPH = 79
EPT = NPH * CPH * CH
EPAD = NS * EPT
RPS = NPAD // NS
DUMMY = NPAD - 1
SLOTS = 6
LAG = 3

_mesh = plsc.VectorSubcoreMesh(core_axis_name="c", subcore_axis_name="s")


def _sc_hop_body(z3, rowt, colt, ypart, row_v, col_v,
                 b0, b1, b2, b3, b4, b5,
                 g0, g1, g2, g3, g4, g5,
                 s0, s1, s2, s3, s4, s5,
                 y_sh):
    c = lax.axis_index("c")
    s = lax.axis_index("s")
    bufs = [b0, b1, b2, b3, b4, b5]
    gsem = [g0, g1, g2, g3, g4, g5]
    ssem = [s0, s1, s2, s3, s4, s5]
    ztab = z3.at[c]

    def drain_gather(t):
        pltpu.make_async_copy(ztab.at[pl.ds(0, CH)], bufs[t], gsem[t]).wait()

    def drain_scatter(t):
        pltpu.make_async_copy(ztab.at[pl.ds(0, CH)],
                              y_sh.at[pl.ds(0, CH)], ssem[t]).wait()

    zero16 = jnp.zeros((16,), jnp.float32)

    @pl.loop(0, CH)
    def _zero(i):
        for g in range(HF // 16):
            b0[i, pl.ds(g * 16, 16)] = zero16

    for q in range(RPS // CH):
        pltpu.sync_copy(b0, y_sh.at[pl.ds(s * RPS + q * CH, CH)])
    plsc.subcore_barrier()

    for ph in range(NPH):
        pltpu.sync_copy(rowt.at[s].at[ph], row_v)
        pltpu.sync_copy(colt.at[s].at[ph], col_v)

        @pl.loop(0, CPH)
        def _edges(j):
            pltpu.sync_copy(ztab.at[row_v.at[j]], b0)
            pltpu.sync_copy(b0, y_sh.at[col_v.at[j]], add=True)

    plsc.subcore_barrier()
    for q in range(RPS // CH):
        off = s * RPS + q * CH
        pltpu.sync_copy(y_sh.at[pl.ds(off, CH)], b1)
        pltpu.sync_copy(b1, ypart.at[c].at[pl.ds(off, CH)])


_sc_hop = pl.kernel(
    _sc_hop_body,
    out_type=jax.ShapeDtypeStruct((NC, NPAD, HF), jnp.float32),
    mesh=_mesh,
    compiler_params=pltpu.CompilerParams(use_tc_tiling_on_sc=False),
    scratch_types=(
        [
            pltpu.VMEM((CPH, CH), jnp.int32),
            pltpu.VMEM((CPH, CH), jnp.int32),
        ]
        + [pltpu.VMEM((CH, HF), jnp.float32)] * SLOTS
        + [pltpu.SemaphoreType.DMA] * (2 * SLOTS)
        + [pltpu.VMEM_SHARED((NPAD, HF), jnp.float32)]
    ),
)


_RB = 1280
_GRID = NPAD // _RB


def _prep_body(dp, x4st, dvb, dv2b, h0, z0):
    b = pl.program_id(0)
    deg = dp[...][:, 0:1] + 1.0
    dv = lax.rsqrt(deg)
    ridx = lax.broadcasted_iota(jnp.int32, (_RB, 1), 0) + b * _RB
    dv = jnp.where(ridx < N, dv, 0.0)
    dvb[...] = jnp.broadcast_to(dv, (_RB, HF))
    dv2b[...] = jnp.broadcast_to(dv * dv, (_RB, HF))
    xb = x4st[...][0]
    h0[0] = ALPHA[0] * xb
    z0[0] = dv * xb


_prep = pl.pallas_call(
    _prep_body,
    grid=(_GRID, NC),
    in_specs=[
        pl.BlockSpec((_RB, HF), lambda b, hb: (b, 0)),
        pl.BlockSpec((1, _RB, HF), lambda b, hb: (hb, b, 0)),
    ],
    out_specs=[
        pl.BlockSpec((_RB, HF), lambda b, hb: (b, 0)),
        pl.BlockSpec((_RB, HF), lambda b, hb: (b, 0)),
        pl.BlockSpec((1, _RB, HF), lambda b, hb: (hb, b, 0)),
        pl.BlockSpec((1, _RB, HF), lambda b, hb: (hb, b, 0)),
    ],
    out_shape=[
        jax.ShapeDtypeStruct((NPAD, HF), jnp.float32),
        jax.ShapeDtypeStruct((NPAD, HF), jnp.float32),
        jax.ShapeDtypeStruct((NC, NPAD, HF), jnp.float32),
        jax.ShapeDtypeStruct((NC, NPAD, HF), jnp.float32),
    ],
)


def _make_combine(cst):
    def body(yp, z, h, dvb, dv2b, ho, zo):
        yt = yp[...][0] + z[...][0]
        ho[0] = h[...][0] + cst * (dvb[...] * yt)
        zo[0] = dv2b[...] * yt

    st = lambda: pl.BlockSpec((1, _RB, HF), lambda b, hb: (hb, b, 0))
    fl = lambda: pl.BlockSpec((_RB, HF), lambda b, hb: (b, 0))
    return pl.pallas_call(
        body,
        grid=(_GRID, NC),
        in_specs=[st(), st(), st(), fl(), fl()],
        out_specs=[st(), st()],
        out_shape=[jax.ShapeDtypeStruct((NC, NPAD, HF), jnp.float32)] * 2,
    )


def _make_layer_end(alpha_next, has_res):
    def body(*refs):
        if has_res:
            (hA, hB, W11, W21, W12, W22, bv1, bv2, rA, rB, dvb,
             xA, xB, hA2, hB2, zA2, zB2) = refs
        else:
            (hA, hB, W11, W21, W12, W22, bv1, bv2, dvb,
             xA, xB, hA2, hB2, zA2, zB2) = refs
        a = hA[...]
        bb = hB[...]
        vA = jnp.dot(a, W11[...], preferred_element_type=jnp.float32) \
            + jnp.dot(bb, W21[...], preferred_element_type=jnp.float32)
        vB = jnp.dot(a, W12[...], preferred_element_type=jnp.float32) \
            + jnp.dot(bb, W22[...], preferred_element_type=jnp.float32)
        vA = jnp.maximum(vA + bv1[...], 0.0)
        vB = jnp.maximum(vB + bv2[...], 0.0)
        if has_res:
            vA = vA + rA[...]
            vB = vB + rB[...]
        dv = dvb[...]
        xA[...] = vA
        xB[...] = vB
        hA2[...] = alpha_next * vA
        hB2[...] = alpha_next * vB
        zA2[...] = dv * vA
        zB2[...] = dv * vB

    half = lambda: pl.BlockSpec((_RB, HF), lambda b: (b, 0))
    wblk = lambda: pl.BlockSpec((HF, HF), lambda b: (0, 0))
    bblk = lambda: pl.BlockSpec((1, HF), lambda b: (0, 0))
    in_specs = [half(), half(), wblk(), wblk(), wblk(), wblk(), bblk(), bblk()]
    if has_res:
        in_specs += [half(), half()]
    in_specs += [half()]
    return pl.pallas_call(
        body,
        grid=(_GRID,),
        in_specs=in_specs,
        out_specs=[half() for _ in range(6)],
        out_shape=[jax.ShapeDtypeStruct((NPAD, HF), jnp.float32)] * 6,
    )


def _head_body(hA, hB, WmA, WmB, bm, pvec, o):
    hm = jnp.dot(hA[...], WmA[...], preferred_element_type=jnp.float32) \
        + jnp.dot(hB[...], WmB[...], preferred_element_type=jnp.float32) \
        + bm[...]
    pv = pvec[...]
    pn = pv * lax.rsqrt(jnp.sum(pv * pv))
    s = jnp.tanh(jnp.sum(hm * pn, axis=1, keepdims=True))
    ridx = lax.broadcasted_iota(jnp.int32, (NPAD, 1), 0)
    s = jnp.where(ridx < N, s, -jnp.inf)
    acc0 = jnp.full((1, 128), -jnp.inf, jnp.float32)

    def it(_, carry):
        sc, acc = carry
        m = jnp.max(sc)
        mask = sc == m
        contrib = jnp.max(jnp.where(mask, m * hm, -jnp.inf), axis=0,
                          keepdims=True)
        acc = jnp.maximum(acc, contrib)
        sc = jnp.where(mask, -jnp.inf, sc)
        return sc, acc

    _, acc = lax.fori_loop(0, 30, it, (s, acc0))
    o[...] = acc


_head = pl.pallas_call(
    _head_body,
    out_shape=jax.ShapeDtypeStruct((1, 128), jnp.float32),
)


def kernel(x, edge_index, W0, b0, W1, b1, W2, b2, W3, b3, W4, b4, Wm, bm, p):
    row = edge_index[0].astype(jnp.int32)
    col = edge_index[1].astype(jnp.int32)
    fill = jnp.full((EPAD - E,), DUMMY, jnp.int32)
    rowt = jnp.concatenate([row, fill]).reshape(NS, NPH, CPH, CH)
    colt = jnp.concatenate([col, fill]).reshape(NS, NPH, CPH, CH)

    ones3 = jnp.ones((NC, NPAD, HF), jnp.float32)
    degp = _sc_hop(ones3, rowt, colt)
    x4A = jnp.pad(x[:, :4], ((0, NPAD - N), (0, HF - 4)))
    x4st = jnp.stack([x4A, jnp.zeros_like(x4A)])
    dvb, dv2b, hst, zst = _prep(degp[0], x4st)

    W0p = jnp.pad(W0, ((0, 124), (0, 0)))
    Ws = [W0p, W1, W2, W3, W4]
    bvs = [b0, b1, b2, b3, b4]

    def run_layer(hst, zst, comb):
        def step(_, hz):
            h, z = hz
            yp = _sc_hop(z, rowt, colt)
            h, z = comb(yp, z, h, dvb, dv2b)
            return h, z

        return lax.fori_loop(0, K, step, (hst, zst))

    xs = [None]
    for li in range(5):
        hst, zst = run_layer(hst, zst, _make_combine((1 - ALPHA[li]) / K))
        W = Ws[li]
        parts = (W[:HF, :HF], W[HF:, :HF], W[:HF, HF:], W[HF:, HF:])
        bv1 = bvs[li][:HF].reshape(1, HF)
        bv2 = bvs[li][HF:].reshape(1, HF)
        alpha_next = ALPHA[li + 1] if li < 4 else 0.0
        has_res = li >= 2
        args = [hst[0], hst[1], *parts, bv1, bv2]
        if has_res:
            args += [xs[li][0], xs[li][1]]
        args += [dvb]
        xA, xB, hA2, hB2, zA2, zB2 = _make_layer_end(alpha_next, has_res)(*args)
        hst = jnp.stack([hA2, hB2])
        zst = jnp.stack([zA2, zB2])
        xs.append((xA, xB))

    return _head(xs[5][0], xs[5][1], Wm[:HF, :], Wm[HF:, :],
                 bm.reshape(1, 128), p.reshape(1, 128))


# speedup vs baseline: 8.4647x; 1.3201x over previous
import jax
import jax.numpy as jnp
from jax import lax
from jax.experimental import pallas as pl
from jax.experimental.pallas import tpu as pltpu
from jax.experimental.pallas import tpu_sc as plsc

N = 10000
E = 320000
K = 20
ALPHA = [0.7, 0.7, 0.7 / 2.0, 0.7 / 3.0, 0.7 / 4.0]

NPAD = 10240
NC, NS = 2, 16
CH = 128
HF = 64
NPH = 2
CPH = 79
EPT = NPH * CPH * CH
EPAD = NS * EPT
RPS = NPAD // NS
DUMMY = NPAD - 1
SLOTS = 6
LAG = 3

_mesh = plsc.VectorSubcoreMesh(core_axis_name="c", subcore_axis_name="s")


def _sc_hop_body(z3, rowt, colt, ypart, row_v, col_v,
                 b0, b1, b2, b3, b4, b5,
                 g0, g1, g2, g3, g4, g5,
                 s0, s1, s2, s3, s4, s5,
                 y_sh):
    c = lax.axis_index("c")
    s = lax.axis_index("s")
    bufs = [b0, b1, b2, b3, b4, b5]
    gsem = [g0, g1, g2, g3, g4, g5]
    ssem = [s0, s1, s2, s3, s4, s5]
    ztab = z3.at[c]

    def drain_gather(t):
        pltpu.make_async_copy(ztab.at[pl.ds(0, CH)], bufs[t], gsem[t]).wait()

    def drain_scatter(t):
        pltpu.make_async_copy(bufs[t], y_sh.at[pl.ds(0, CH)], ssem[t]).wait()

    zero16 = jnp.zeros((16,), jnp.float32)

    @pl.loop(0, CH)
    def _zero(i):
        for g in range(HF // 16):
            b0[i, pl.ds(g * 16, 16)] = zero16

    for q in range(RPS // CH):
        pltpu.sync_copy(b0, y_sh.at[pl.ds(s * RPS + q * CH, CH)])
    plsc.subcore_barrier()

    for ph in range(NPH):
        pltpu.sync_copy(rowt.at[s].at[ph], row_v)
        pltpu.sync_copy(colt.at[s].at[ph], col_v)

        @pl.loop(0, (CPH + LAG + SLOTS - 1) // SLOTS + 1)
        def _grp(g):
            for t in range(SLOTS):
                j = g * SLOTS + t

                @pl.when(j <= CPH - 1)
                def _gather():
                    @pl.when(j >= SLOTS)
                    def _w():
                        drain_scatter(t)

                    pltpu.async_copy(ztab.at[row_v.at[j]], bufs[t], gsem[t])

                i = j - LAG
                u = (t - LAG) % SLOTS

                @pl.when(jnp.logical_and(i >= 0, i <= CPH - 1))
                def _scatter():
                    drain_gather(u)
                    pltpu.async_copy(bufs[u], y_sh.at[col_v.at[i]],
                                     ssem[u], add=True)

        for t in range(SLOTS):
            drain_scatter(t)

    plsc.subcore_barrier()
    for q in range(RPS // CH):
        off = s * RPS + q * CH
        pltpu.sync_copy(y_sh.at[pl.ds(off, CH)], b1)
        pltpu.sync_copy(b1, ypart.at[c].at[pl.ds(off, CH)])


_sc_hop = pl.kernel(
    _sc_hop_body,
    out_type=jax.ShapeDtypeStruct((NC, NPAD, HF), jnp.float32),
    mesh=_mesh,
    compiler_params=pltpu.CompilerParams(use_tc_tiling_on_sc=False),
    scratch_types=(
        [
            pltpu.VMEM((CPH, CH), jnp.int32),
            pltpu.VMEM((CPH, CH), jnp.int32),
        ]
        + [pltpu.VMEM((CH, HF), jnp.float32)] * SLOTS
        + [pltpu.SemaphoreType.DMA] * (2 * SLOTS)
        + [pltpu.VMEM_SHARED((NPAD, HF), jnp.float32)]
    ),
)


_RB = 1280
_GRID = NPAD // _RB


def _prep_body(dp, x4st, dvb, dv2b, dvbf, h0, z0):
    b = pl.program_id(0)
    deg = dp[...][:, 0:1] + 1.0
    dv = lax.rsqrt(deg)
    ridx = lax.broadcasted_iota(jnp.int32, (_RB, 1), 0) + b * _RB
    dv = jnp.where(ridx < N, dv, 0.0)
    dvb[...] = jnp.broadcast_to(dv, (_RB, HF))
    dv2b[...] = jnp.broadcast_to(dv * dv, (_RB, HF))
    dvbf[...] = jnp.broadcast_to(dv, (_RB, 128))
    xb = x4st[...][0]
    h0[0] = ALPHA[0] * xb
    z0[0] = dv * xb


_prep = pl.pallas_call(
    _prep_body,
    grid=(_GRID, NC),
    in_specs=[
        pl.BlockSpec((_RB, HF), lambda b, hb: (b, 0)),
        pl.BlockSpec((1, _RB, HF), lambda b, hb: (hb, b, 0)),
    ],
    out_specs=[
        pl.BlockSpec((_RB, HF), lambda b, hb: (b, 0)),
        pl.BlockSpec((_RB, HF), lambda b, hb: (b, 0)),
        pl.BlockSpec((_RB, 128), lambda b, hb: (b, 0)),
        pl.BlockSpec((1, _RB, HF), lambda b, hb: (hb, b, 0)),
        pl.BlockSpec((1, _RB, HF), lambda b, hb: (hb, b, 0)),
    ],
    out_shape=[
        jax.ShapeDtypeStruct((NPAD, HF), jnp.float32),
        jax.ShapeDtypeStruct((NPAD, HF), jnp.float32),
        jax.ShapeDtypeStruct((NPAD, 128), jnp.float32),
        jax.ShapeDtypeStruct((NC, NPAD, HF), jnp.float32),
        jax.ShapeDtypeStruct((NC, NPAD, HF), jnp.float32),
    ],
)


def _make_combine(cst):
    def body(yp, z, h, dvb, dv2b, ho, zo):
        yt = yp[...][0] + z[...][0]
        ho[0] = h[...][0] + cst * (dvb[...] * yt)
        zo[0] = dv2b[...] * yt

    st = lambda: pl.BlockSpec((1, _RB, HF), lambda b, hb: (hb, b, 0))
    fl = lambda: pl.BlockSpec((_RB, HF), lambda b, hb: (b, 0))
    return pl.pallas_call(
        body,
        grid=(_GRID, NC),
        in_specs=[st(), st(), st(), fl(), fl()],
        out_specs=[st(), st()],
        out_shape=[jax.ShapeDtypeStruct((NC, NPAD, HF), jnp.float32)] * 2,
    )


def _make_layer_end(alpha_next, has_res):
    def body(*refs):
        if has_res:
            h, W, bvec, res, dvb, xo, ho, zo = refs
        else:
            h, W, bvec, dvb, xo, ho, zo = refs
        v = jnp.dot(h[...], W[...], preferred_element_type=jnp.float32)
        v = jnp.maximum(v + bvec[...], 0.0)
        if has_res:
            v = v + res[...]
        xo[...] = v
        ho[...] = alpha_next * v
        zo[...] = dvb[...] * v

    full = lambda: pl.BlockSpec((_RB, 128), lambda b: (b, 0))
    in_specs = [
        full(),
        pl.BlockSpec((128, 128), lambda b: (0, 0)),
        pl.BlockSpec((1, 128), lambda b: (0, 0)),
    ]
    if has_res:
        in_specs.append(full())
    in_specs.append(full())
    return pl.pallas_call(
        body,
        grid=(_GRID,),
        in_specs=in_specs,
        out_specs=[full() for _ in range(3)],
        out_shape=[jax.ShapeDtypeStruct((NPAD, 128), jnp.float32)] * 3,
    )


def _head_body(h, Wm, bm, pcol, prow, o):
    hm = jnp.dot(h[...], Wm[...], preferred_element_type=jnp.float32) + bm[...]
    sraw = jnp.dot(hm, pcol[...], preferred_element_type=jnp.float32)
    pv = prow[...]
    s = jnp.tanh(sraw / jnp.sqrt(jnp.sum(pv * pv)))
    ridx = lax.broadcasted_iota(jnp.int32, (NPAD, 1), 0)
    s = jnp.where(ridx < N, s, -jnp.inf)
    acc0 = jnp.full((1, 128), -jnp.inf, jnp.float32)

    def it(_, carry):
        sc, acc = carry
        m = jnp.max(sc)
        mask = sc == m
        contrib = jnp.max(jnp.where(mask, m * hm, -jnp.inf), axis=0,
                          keepdims=True)
        acc = jnp.maximum(acc, contrib)
        sc = jnp.where(mask, -jnp.inf, sc)
        return sc, acc

    _, acc = lax.fori_loop(0, 30, it, (s, acc0))
    o[...] = acc


_head = pl.pallas_call(
    _head_body,
    out_shape=jax.ShapeDtypeStruct((1, 128), jnp.float32),
)


def kernel(x, edge_index, W0, b0, W1, b1, W2, b2, W3, b3, W4, b4, Wm, bm, p):
    row = edge_index[0].astype(jnp.int32)
    col = edge_index[1].astype(jnp.int32)
    fill = jnp.full((EPAD - E,), DUMMY, jnp.int32)
    rowt = jnp.concatenate([row, fill]).reshape(NS, NPH, CPH, CH)
    colt = jnp.concatenate([col, fill]).reshape(NS, NPH, CPH, CH)

    ones3 = jnp.ones((NC, NPAD, HF), jnp.float32)
    degp = _sc_hop(ones3, rowt, colt)
    x4A = jnp.pad(x[:, :4], ((0, NPAD - N), (0, HF - 4)))
    x4st = jnp.stack([x4A, jnp.zeros_like(x4A)])
    dvb, dv2b, dvbf, hst, zst = _prep(degp[0], x4st)

    W0p = jnp.pad(W0, ((0, 124), (0, 0)))
    Ws = [W0p, W1, W2, W3, W4]
    bvs = [b0, b1, b2, b3, b4]

    def run_layer(hst, zst, comb):
        def step(_, hz):
            h, z = hz
            yp = _sc_hop(z, rowt, colt)
            h, z = comb(yp, z, h, dvb, dv2b)
            return h, z

        return lax.fori_loop(0, K, step, (hst, zst))

    def split(a):
        return jnp.stack([a[:, :HF], a[:, HF:]])

    xs = [None]
    for li in range(5):
        hst, zst = run_layer(hst, zst, _make_combine((1 - ALPHA[li]) / K))
        hfull = jnp.concatenate([hst[0], hst[1]], axis=1)
        alpha_next = ALPHA[li + 1] if li < 4 else 0.0
        has_res = li >= 2
        args = [hfull, Ws[li], bvs[li].reshape(1, 128)]
        if has_res:
            args += [xs[li]]
        args += [dvbf]
        xn, hn, zn = _make_layer_end(alpha_next, has_res)(*args)
        hst = split(hn)
        zst = split(zn)
        xs.append(xn)

    return _head(xs[5], Wm, bm.reshape(1, 128), p.reshape(128, 1),
                 p.reshape(1, 128))
